# baseline (speedup 1.0000x reference)
import jax
import jax.numpy as jnp
from jax import lax
from jax.experimental import pallas as pl
from jax.experimental.pallas import tpu as pltpu
from jax.experimental.pallas import tpu_sc as plsc

_N = 10000
_E = 320000
_D = 128
_H = 16
_C = 16

_L = 16
_NT = 16
_RPT = 640
_N1 = _NT * _RPT
_B = 128
_NBATCH = 162
_EPT = _NBATCH * _B
_E1 = _NT * _EPT
_W = 2 * _H


def _rsqrt_newton(v):
    v = jnp.maximum(v, jnp.float32(1e-24))
    i = plsc.bitcast(v, jnp.int32)
    i = jnp.int32(0x5F3759DF) - lax.shift_right_logical(i, 1)
    y = plsc.bitcast(i, jnp.float32)
    for _ in range(3):
        y = y * (jnp.float32(1.5) - jnp.float32(0.5) * v * y * y)
    return y


def _conv_body(h_hbm, srcs_hbm, dsts_hbm, beta_hbm, out_hbm, tbl_hbm,
               src_t, dst_t, hbuf, nbuf, cbuf,
               rows_s0, rows_d0, rows_s1, rows_d1,
               out_rows0, ab0, out_rows1, ab1,
               rbuf, dbuf, beta_v,
               denom_spm, num_spm):
    w = lax.axis_index("s")
    base = w * _RPT
    lane = lax.iota(jnp.int32, _L)
    zrow = jnp.zeros((_L,), jnp.float32)

    pltpu.sync_copy(srcs_hbm.at[w], src_t)
    pltpu.sync_copy(dsts_hbm.at[w], dst_t)
    pltpu.sync_copy(beta_hbm, beta_v)

    pltpu.sync_copy(h_hbm.at[pl.ds(base, _RPT)], hbuf)

    def p0(rg, carry):
        acc = zrow
        for u in range(_L):
            r = rg * _L + u
            hr = hbuf[r, :]
            acc = jnp.where(lane == u, jnp.sum(hr * hr), acc)
        rinv = _rsqrt_newton(acc)
        for u in range(_L):
            r = rg * _L + u
            hr = hbuf[r, :]
            cbuf[r, 0:_H] = hr * rinv[u]
            cbuf[r, _H:_W] = hr
            nbuf[r, :] = zrow
        rbuf[pl.ds(rg * _L, _L)] = rinv
        dbuf[pl.ds(rg * _L, _L)] = zrow
        return carry

    lax.fori_loop(0, _RPT // _L, p0, None)
    pltpu.sync_copy(cbuf, tbl_hbm.at[pl.ds(base, _RPT)])
    pltpu.sync_copy(dbuf, denom_spm.at[pl.ds(base, _RPT)])
    pltpu.sync_copy(nbuf, num_spm.at[pl.ds(base, _RPT)])
    plsc.subcore_barrier()

    bufs = ((rows_s0, rows_d0, out_rows0, ab0),
            (rows_s1, rows_d1, out_rows1, ab1))
    bv = beta_v[...]

    def issue_gathers(i, p, sem):
        rs, rd, _, _ = bufs[p]
        return (pltpu.async_copy(tbl_hbm.at[src_t.at[i]], rs, sem),
                pltpu.async_copy(tbl_hbm.at[dst_t.at[i]], rd, sem))

    def process(i, p):
        rs, rd, orows, ab = bufs[p]

        def grp(g, c):
            acc = zrow
            for u in range(_L):
                e = g * _L + u
                acc = jnp.where(lane == u,
                                jnp.sum(rs[e, 0:_H] * rd[e, 0:_H]), acc)
            off = g * _L
            av = jnp.exp(acc * bv)
            ab[pl.ds(off, _L)] = av
            for u in range(_L):
                e = g * _L + u
                orows[e, :] = rs[e, _H:_W] * av[u]
            return c

        lax.fori_loop(0, _B // _L, grp, None)

    def issue_scatters(i, p, sem):
        _, _, orows, ab = bufs[p]
        di = dst_t.at[i]
        return (pltpu.async_copy(ab, denom_spm.at[di], sem, add=True),
                pltpu.async_copy(orows, num_spm.at[di], sem, add=True))

    def p1_scoped(g0, g1, s0, s1):
        def p1(j, carry):
            b0 = j * 2
            dg0 = issue_gathers(b0, 0, g0)
            dg1 = issue_gathers(b0 + 1, 1, g1)
            for d in dg0:
                d.wait()
            process(b0, 0)
            ds0 = issue_scatters(b0, 0, s0)
            for d in dg1:
                d.wait()
            process(b0 + 1, 1)
            ds1 = issue_scatters(b0 + 1, 1, s1)
            for d in ds0:
                d.wait()
            for d in ds1:
                d.wait()
            return carry

        lax.fori_loop(0, _NBATCH // 2, p1, None)

    pl.run_scoped(p1_scoped,
                  g0=pltpu.SemaphoreType.DMA(()),
                  g1=pltpu.SemaphoreType.DMA(()),
                  s0=pltpu.SemaphoreType.DMA(()),
                  s1=pltpu.SemaphoreType.DMA(()))
    plsc.subcore_barrier()

    pltpu.sync_copy(num_spm.at[pl.ds(base, _RPT)], nbuf)
    pltpu.sync_copy(denom_spm.at[pl.ds(base, _RPT)], dbuf)

    def p2(rg, carry):
        dv = dbuf[pl.ds(rg * _L, _L)]
        dinv = jnp.float32(1.0) / jnp.maximum(dv, jnp.float32(1e-30))
        for u in range(_L):
            r = rg * _L + u
            hbuf[r, :] = nbuf[r, :] * dinv[u]
        return carry

    lax.fori_loop(0, _RPT // _L, p2, None)
    pltpu.sync_copy(hbuf, out_hbm.at[pl.ds(base, _RPT)])


_conv = pl.kernel(
    _conv_body,
    out_type=(
        jax.ShapeDtypeStruct((_N1, _H), jnp.float32),
        jax.ShapeDtypeStruct((_N1, _W), jnp.float32),
    ),
    mesh=plsc.VectorSubcoreMesh(
        core_axis_name="c", subcore_axis_name="s", num_cores=1
    ),
    compiler_params=pltpu.CompilerParams(
        needs_layout_passes=False, use_tc_tiling_on_sc=False
    ),
    scratch_types=[
        pltpu.VMEM((_NBATCH, _B), jnp.int32),
        pltpu.VMEM((_NBATCH, _B), jnp.int32),
        pltpu.VMEM((_RPT, _H), jnp.float32),
        pltpu.VMEM((_RPT, _H), jnp.float32),
        pltpu.VMEM((_RPT, _W), jnp.float32),
        pltpu.VMEM((_B, _W), jnp.float32),
        pltpu.VMEM((_B, _W), jnp.float32),
        pltpu.VMEM((_B, _W), jnp.float32),
        pltpu.VMEM((_B, _W), jnp.float32),
        pltpu.VMEM((_B, _H), jnp.float32),
        pltpu.VMEM((_B,), jnp.float32),
        pltpu.VMEM((_B, _H), jnp.float32),
        pltpu.VMEM((_B,), jnp.float32),
        pltpu.VMEM((_RPT,), jnp.float32),
        pltpu.VMEM((_RPT,), jnp.float32),
        pltpu.VMEM((_L,), jnp.float32),
        pltpu.VMEM_SHARED((_N1,), jnp.float32),
        pltpu.VMEM_SHARED((_N1, _H), jnp.float32),
    ],
)


def _pre_body(x_ref, w_ref, b_ref, o_ref):
    acc = jnp.dot(x_ref[...], w_ref[...], preferred_element_type=jnp.float32)
    o_ref[...] = jnp.maximum(acc + b_ref[...], jnp.float32(0.0))


_pre = pl.pallas_call(
    _pre_body,
    grid=(10,),
    in_specs=[
        pl.BlockSpec((_N // 10, _D), lambda i: (i, 0)),
        pl.BlockSpec((_D, _H), lambda i: (0, 0)),
        pl.BlockSpec((1, _H), lambda i: (0, 0)),
    ],
    out_specs=pl.BlockSpec((_N // 10, _H), lambda i: (i, 0)),
    out_shape=jax.ShapeDtypeStruct((_N, _H), jnp.float32),
)


def _post_body(h_ref, w_ref, b_ref, o_ref):
    z = jnp.dot(h_ref[...], w_ref[...], preferred_element_type=jnp.float32)
    z = z + b_ref[...]
    z = z - jnp.max(z, axis=1, keepdims=True)
    o_ref[...] = z - jnp.log(jnp.sum(jnp.exp(z), axis=1, keepdims=True))


_post = pl.pallas_call(
    _post_body,
    grid=(10,),
    in_specs=[
        pl.BlockSpec((_N1 // 10, _H), lambda i: (i, 0)),
        pl.BlockSpec((_H, _C), lambda i: (0, 0)),
        pl.BlockSpec((1, _C), lambda i: (0, 0)),
    ],
    out_specs=pl.BlockSpec((_N1 // 10, _C), lambda i: (i, 0)),
    out_shape=jax.ShapeDtypeStruct((_N1, _C), jnp.float32),
)


def kernel(x, edge_index, W1, b1, beta2, beta3, beta4, W2, b2):
    h0 = _pre(x, W1, b1.reshape(1, _H))
    hp = jnp.concatenate(
        [h0, jnp.zeros((_N1 - _N, _H), jnp.float32)], axis=0
    )

    src = edge_index[0].astype(jnp.int32)
    dst = edge_index[1].astype(jnp.int32)
    loop = jnp.arange(_N, dtype=jnp.int32)
    pad = jnp.full((_E1 - _E - _N,), _N, dtype=jnp.int32)
    srcs = jnp.concatenate([src, loop, pad]).reshape(_NT, _NBATCH, _B)
    dsts = jnp.concatenate([dst, loop, pad]).reshape(_NT, _NBATCH, _B)

    ones = jnp.ones((_L,), jnp.float32)
    h, _unused = _conv(hp, srcs, dsts, ones)
    h, _unused = _conv(h, srcs, dsts, ones * beta2)
    h, _unused = _conv(h, srcs, dsts, ones * beta3)
    h, _unused = _conv(h, srcs, dsts, ones * beta4)

    out = _post(h, W2, b2.reshape(1, _C))
    return out[:_N]


# speedup vs baseline: 1.0678x; 1.0678x over previous
import jax
import jax.numpy as jnp
from jax import lax
from jax.experimental import pallas as pl
from jax.experimental.pallas import tpu as pltpu
from jax.experimental.pallas import tpu_sc as plsc

_N = 10000
_E = 320000
_D = 128
_H = 16
_C = 16

_L = 16
_NC = 2
_NT = 16
_NW = _NC * _NT
_RPT = 640
_N1 = _NT * _RPT
_B = 128
_NBATCH = 82
_EPW = _NBATCH * _B
_E1 = _NW * _EPW
_W = 2 * _H


def _rsqrt_newton(v):
    v = jnp.maximum(v, jnp.float32(1e-24))
    i = plsc.bitcast(v, jnp.int32)
    i = jnp.int32(0x5F3759DF) - lax.shift_right_logical(i, 1)
    y = plsc.bitcast(i, jnp.float32)
    for _ in range(3):
        y = y * (jnp.float32(1.5) - jnp.float32(0.5) * v * y * y)
    return y


def _conv_body(np_hbm, dp_hbm, srcs_hbm, dsts_hbm, beta_hbm,
               npo_hbm, dpo_hbm, tbl_hbm,
               src_t, dst_t, nbuf, n2buf, cbuf,
               rows_s0, rows_d0, rows_s1, rows_d1,
               out_rows0, ab0, out_rows1, ab1,
               dbuf, d2buf, beta_v,
               denom_spm, num_spm):
    c = lax.axis_index("c")
    s = lax.axis_index("s")
    wid = c * _NT + s
    base = s * _RPT
    lane = lax.iota(jnp.int32, _L)
    zrow = jnp.zeros((_L,), jnp.float32)

    pltpu.sync_copy(srcs_hbm.at[wid], src_t)
    pltpu.sync_copy(dsts_hbm.at[wid], dst_t)
    pltpu.sync_copy(beta_hbm, beta_v)

    pltpu.sync_copy(np_hbm.at[0, pl.ds(base, _RPT)], nbuf)
    pltpu.sync_copy(np_hbm.at[1, pl.ds(base, _RPT)], n2buf)
    pltpu.sync_copy(dp_hbm.at[0, pl.ds(base, _RPT)], dbuf)
    pltpu.sync_copy(dp_hbm.at[1, pl.ds(base, _RPT)], d2buf)

    def p0(rg, carry):
        off = rg * _L
        dv = dbuf[pl.ds(off, _L)] + d2buf[pl.ds(off, _L)]
        dinv = jnp.float32(1.0) / jnp.maximum(dv, jnp.float32(1e-30))
        acc = zrow
        for u in range(_L):
            r = off + u
            hr = (nbuf[r, :] + n2buf[r, :]) * dinv[u]
            cbuf[r, _H:_W] = hr
            acc = jnp.where(lane == u, jnp.sum(hr * hr), acc)
            nbuf[r, :] = zrow
        rinv = _rsqrt_newton(acc)
        for u in range(_L):
            r = off + u
            cbuf[r, 0:_H] = cbuf[r, _H:_W] * rinv[u]
        dbuf[pl.ds(off, _L)] = zrow
        return carry

    lax.fori_loop(0, _RPT // _L, p0, None)
    pltpu.sync_copy(cbuf, tbl_hbm.at[pl.ds(base, _RPT)])
    pltpu.sync_copy(dbuf, denom_spm.at[pl.ds(base, _RPT)])
    pltpu.sync_copy(nbuf, num_spm.at[pl.ds(base, _RPT)])
    plsc.subcore_barrier()

    bufs = ((rows_s0, rows_d0, out_rows0, ab0),
            (rows_s1, rows_d1, out_rows1, ab1))
    bv = beta_v[...]

    def issue_gathers(i, p, sem):
        rs, rd, _, _ = bufs[p]
        return (pltpu.async_copy(tbl_hbm.at[src_t.at[i]], rs, sem),
                pltpu.async_copy(tbl_hbm.at[dst_t.at[i]], rd, sem))

    def process(i, p):
        rs, rd, orows, ab = bufs[p]

        def grp(g, carry):
            acc = zrow
            for u in range(_L):
                e = g * _L + u
                acc = jnp.where(lane == u,
                                jnp.sum(rs[e, 0:_H] * rd[e, 0:_H]), acc)
            off = g * _L
            av = jnp.exp(acc * bv)
            ab[pl.ds(off, _L)] = av
            for u in range(_L):
                e = g * _L + u
                orows[e, :] = rs[e, _H:_W] * av[u]
            return carry

        lax.fori_loop(0, _B // _L, grp, None)

    def issue_scatters(i, p, sem):
        _, _, orows, ab = bufs[p]
        di = dst_t.at[i]
        return (pltpu.async_copy(ab, denom_spm.at[di], sem, add=True),
                pltpu.async_copy(orows, num_spm.at[di], sem, add=True))

    def p1_scoped(g0, g1, s0, s1):
        def p1(j, carry):
            b0 = j * 2
            dg0 = issue_gathers(b0, 0, g0)
            dg1 = issue_gathers(b0 + 1, 1, g1)
            for d in dg0:
                d.wait()
            process(b0, 0)
            ds0 = issue_scatters(b0, 0, s0)
            for d in dg1:
                d.wait()
            process(b0 + 1, 1)
            ds1 = issue_scatters(b0 + 1, 1, s1)
            for d in ds0:
                d.wait()
            for d in ds1:
                d.wait()
            return carry

        lax.fori_loop(0, _NBATCH // 2, p1, None)

    pl.run_scoped(p1_scoped,
                  g0=pltpu.SemaphoreType.DMA(()),
                  g1=pltpu.SemaphoreType.DMA(()),
                  s0=pltpu.SemaphoreType.DMA(()),
                  s1=pltpu.SemaphoreType.DMA(()))
    plsc.subcore_barrier()

    pltpu.sync_copy(num_spm.at[pl.ds(base, _RPT)], nbuf)
    pltpu.sync_copy(denom_spm.at[pl.ds(base, _RPT)], dbuf)
    pltpu.sync_copy(nbuf, npo_hbm.at[c, pl.ds(base, _RPT)])
    pltpu.sync_copy(dbuf, dpo_hbm.at[c, pl.ds(base, _RPT)])


_conv = pl.kernel(
    _conv_body,
    out_type=(
        jax.ShapeDtypeStruct((_NC, _N1, _H), jnp.float32),
        jax.ShapeDtypeStruct((_NC, _N1), jnp.float32),
        jax.ShapeDtypeStruct((_N1, _W), jnp.float32),
    ),
    mesh=plsc.VectorSubcoreMesh(
        core_axis_name="c", subcore_axis_name="s", num_cores=_NC
    ),
    compiler_params=pltpu.CompilerParams(
        needs_layout_passes=False, use_tc_tiling_on_sc=False
    ),
    scratch_types=[
        pltpu.VMEM((_NBATCH, _B), jnp.int32),
        pltpu.VMEM((_NBATCH, _B), jnp.int32),
        pltpu.VMEM((_RPT, _H), jnp.float32),
        pltpu.VMEM((_RPT, _H), jnp.float32),
        pltpu.VMEM((_RPT, _W), jnp.float32),
        pltpu.VMEM((_B, _W), jnp.float32),
        pltpu.VMEM((_B, _W), jnp.float32),
        pltpu.VMEM((_B, _W), jnp.float32),
        pltpu.VMEM((_B, _W), jnp.float32),
        pltpu.VMEM((_B, _H), jnp.float32),
        pltpu.VMEM((_B,), jnp.float32),
        pltpu.VMEM((_B, _H), jnp.float32),
        pltpu.VMEM((_B,), jnp.float32),
        pltpu.VMEM((_RPT,), jnp.float32),
        pltpu.VMEM((_RPT,), jnp.float32),
        pltpu.VMEM((_L,), jnp.float32),
        pltpu.VMEM_SHARED((_N1,), jnp.float32),
        pltpu.VMEM_SHARED((_N1, _H), jnp.float32),
    ],
)


def _pre_body(x_ref, w_ref, b_ref, o_ref):
    acc = jnp.dot(x_ref[...], w_ref[...], preferred_element_type=jnp.float32)
    o_ref[...] = jnp.maximum(acc + b_ref[...], jnp.float32(0.0))


_pre = pl.pallas_call(
    _pre_body,
    grid=(10,),
    in_specs=[
        pl.BlockSpec((_N // 10, _D), lambda i: (i, 0)),
        pl.BlockSpec((_D, _H), lambda i: (0, 0)),
        pl.BlockSpec((1, _H), lambda i: (0, 0)),
    ],
    out_specs=pl.BlockSpec((_N // 10, _H), lambda i: (i, 0)),
    out_shape=jax.ShapeDtypeStruct((_N, _H), jnp.float32),
)


def _post_body(n_ref, d_ref, w_ref, b_ref, o_ref):
    nsum = n_ref[0] + n_ref[1]
    dsum = d_ref[0] + d_ref[1]
    h = nsum * (jnp.float32(1.0)
                / jnp.maximum(dsum, jnp.float32(1e-30)))[:, None]
    z = jnp.dot(h, w_ref[...], preferred_element_type=jnp.float32)
    z = z + b_ref[...]
    z = z - jnp.max(z, axis=1, keepdims=True)
    o_ref[...] = z - jnp.log(jnp.sum(jnp.exp(z), axis=1, keepdims=True))


_post = pl.pallas_call(
    _post_body,
    grid=(10,),
    in_specs=[
        pl.BlockSpec((_NC, _N1 // 10, _H), lambda i: (0, i, 0)),
        pl.BlockSpec((_NC, _N1 // 10), lambda i: (0, i)),
        pl.BlockSpec((_H, _C), lambda i: (0, 0)),
        pl.BlockSpec((1, _C), lambda i: (0, 0)),
    ],
    out_specs=pl.BlockSpec((_N1 // 10, _C), lambda i: (i, 0)),
    out_shape=jax.ShapeDtypeStruct((_N1, _C), jnp.float32),
)


def kernel(x, edge_index, W1, b1, beta2, beta3, beta4, W2, b2):
    h0 = _pre(x, W1, b1.reshape(1, _H))
    hp = jnp.concatenate(
        [h0, jnp.zeros((_N1 - _N, _H), jnp.float32)], axis=0
    )
    nparts = jnp.stack([hp, jnp.zeros_like(hp)])
    dparts = jnp.stack(
        [jnp.ones((_N1,), jnp.float32), jnp.zeros((_N1,), jnp.float32)]
    )

    src = edge_index[0].astype(jnp.int32)
    dst = edge_index[1].astype(jnp.int32)
    loop = jnp.arange(_N, dtype=jnp.int32)
    pad = jnp.full((_E1 - _E - _N,), _N, dtype=jnp.int32)
    srcs = jnp.concatenate([src, loop, pad]).reshape(_NW, _NBATCH, _B)
    dsts = jnp.concatenate([dst, loop, pad]).reshape(_NW, _NBATCH, _B)

    ones = jnp.ones((_L,), jnp.float32)
    nparts, dparts, _t = _conv(nparts, dparts, srcs, dsts, ones)
    nparts, dparts, _t = _conv(nparts, dparts, srcs, dsts, ones * beta2)
    nparts, dparts, _t = _conv(nparts, dparts, srcs, dsts, ones * beta3)
    nparts, dparts, _t = _conv(nparts, dparts, srcs, dsts, ones * beta4)

    out = _post(nparts, dparts, W2, b2.reshape(1, _C))
    return out[:_N]


# speedup vs baseline: 1.8411x; 1.7242x over previous
import jax
import jax.numpy as jnp
from jax import lax
from jax.experimental import pallas as pl
from jax.experimental.pallas import tpu as pltpu
from jax.experimental.pallas import tpu_sc as plsc

_N = 10000
_E = 320000
_D = 128
_H = 16
_C = 16

_L = 16
_NC = 2
_NT = 16
_NW = _NC * _NT
_RPT = 640
_N1 = _NT * _RPT
_B = 128
_NBATCH = 82
_EPW = _NBATCH * _B
_E1 = _NW * _EPW
_W = 2 * _H


def _rsqrt_newton(v):
    v = jnp.maximum(v, jnp.float32(1e-24))
    i = plsc.bitcast(v, jnp.int32)
    i = jnp.int32(0x5F3759DF) - lax.shift_right_logical(i, 1)
    y = plsc.bitcast(i, jnp.float32)
    for _ in range(3):
        y = y * (jnp.float32(1.5) - jnp.float32(0.5) * v * y * y)
    return y


def _conv_body(np_hbm, dp_hbm, srcs_hbm, dsts_hbm, beta_hbm,
               npo_hbm, dpo_hbm, tbl_hbm,
               src_t, dst_t, nbuf, n2buf, cbuf,
               rows_s0, rows_d0, rows_s1, rows_d1,
               out_rows0, ab0, out_rows1, ab1,
               dbuf, d2buf, beta_v,
               denom_spm, num_spm):
    c = lax.axis_index("c")
    s = lax.axis_index("s")
    wid = c * _NT + s
    base = s * _RPT
    lane = lax.iota(jnp.int32, _L)
    zrow = jnp.zeros((_L,), jnp.float32)

    pltpu.sync_copy(srcs_hbm.at[wid], src_t)
    pltpu.sync_copy(dsts_hbm.at[wid], dst_t)
    pltpu.sync_copy(beta_hbm, beta_v)

    pltpu.sync_copy(np_hbm.at[0, pl.ds(base, _RPT)], nbuf)
    pltpu.sync_copy(np_hbm.at[1, pl.ds(base, _RPT)], n2buf)
    pltpu.sync_copy(dp_hbm.at[0, pl.ds(base, _RPT)], dbuf)
    pltpu.sync_copy(dp_hbm.at[1, pl.ds(base, _RPT)], d2buf)

    def p0(rg, carry):
        off = rg * _L
        dv = dbuf[pl.ds(off, _L)] + d2buf[pl.ds(off, _L)]
        dinv = jnp.float32(1.0) / jnp.maximum(dv, jnp.float32(1e-30))
        acc = zrow
        for u in range(_L):
            r = off + u
            hr = (nbuf[r, :] + n2buf[r, :]) * dinv[u]
            cbuf[r, _H:_W] = hr
            acc = jnp.where(lane == u, jnp.sum(hr * hr), acc)
            nbuf[r, :] = zrow
        rinv = _rsqrt_newton(acc)
        for u in range(_L):
            r = off + u
            cbuf[r, 0:_H] = cbuf[r, _H:_W] * rinv[u]
        dbuf[pl.ds(off, _L)] = zrow
        return carry

    lax.fori_loop(0, _RPT // _L, p0, None)
    pltpu.sync_copy(cbuf, tbl_hbm.at[pl.ds(base, _RPT)])
    pltpu.sync_copy(dbuf, denom_spm.at[pl.ds(base, _RPT)])
    pltpu.sync_copy(nbuf, num_spm.at[pl.ds(base, _RPT)])
    plsc.subcore_barrier()

    bufs = ((rows_s0, rows_d0, out_rows0, ab0),
            (rows_s1, rows_d1, out_rows1, ab1))
    bv = beta_v[...]

    def issue_gathers(i, p, sem):
        rs, rd, _, _ = bufs[p]
        return (pltpu.async_copy(tbl_hbm.at[src_t.at[i]], rs, sem),
                pltpu.async_copy(tbl_hbm.at[dst_t.at[i]], rd, sem))

    def process(i, p):
        rs, rd, orows, ab = bufs[p]

        def grp(g, carry):
            acc = zrow
            for u in range(_L):
                e = g * _L + u
                acc = jnp.where(lane == u,
                                jnp.sum(rs[e, 0:_H] * rd[e, 0:_H]), acc)
            off = g * _L
            av = jnp.exp(acc * bv)
            ab[pl.ds(off, _L)] = av
            for u in range(_L):
                e = g * _L + u
                orows[e, :] = rs[e, _H:_W] * av[u]
            return carry

        lax.fori_loop(0, _B // _L, grp, None)

    def issue_scatters(i, p, sem):
        _, _, orows, ab = bufs[p]
        di = dst_t.at[i]
        return (pltpu.async_copy(ab, denom_spm.at[di], sem, add=True),
                pltpu.async_copy(orows, num_spm.at[di], sem, add=True))

    def p1_scoped(g0, g1, s0, s1):
        def p1(j, carry):
            b0 = j * 2
            dg0 = issue_gathers(b0, 0, g0)
            dg1 = issue_gathers(b0 + 1, 1, g1)
            for d in dg0:
                d.wait()
            process(b0, 0)
            ds0 = issue_scatters(b0, 0, s0)
            for d in dg1:
                d.wait()
            process(b0 + 1, 1)
            ds1 = issue_scatters(b0 + 1, 1, s1)
            for d in ds0:
                d.wait()
            for d in ds1:
                d.wait()
            return carry

        lax.fori_loop(0, _NBATCH // 2, p1, None)

    pl.run_scoped(p1_scoped,
                  g0=pltpu.SemaphoreType.DMA(()),
                  g1=pltpu.SemaphoreType.DMA(()),
                  s0=pltpu.SemaphoreType.DMA(()),
                  s1=pltpu.SemaphoreType.DMA(()))
    plsc.subcore_barrier()

    pltpu.sync_copy(num_spm.at[pl.ds(base, _RPT)], nbuf)
    pltpu.sync_copy(denom_spm.at[pl.ds(base, _RPT)], dbuf)
    pltpu.sync_copy(nbuf, npo_hbm.at[c, pl.ds(base, _RPT)])
    pltpu.sync_copy(dbuf, dpo_hbm.at[c, pl.ds(base, _RPT)])


_conv = pl.kernel(
    _conv_body,
    out_type=(
        jax.ShapeDtypeStruct((_NC, _N1, _H), jnp.float32),
        jax.ShapeDtypeStruct((_NC, _N1), jnp.float32),
        jax.ShapeDtypeStruct((_N1, _W), jnp.float32),
    ),
    mesh=plsc.VectorSubcoreMesh(
        core_axis_name="c", subcore_axis_name="s", num_cores=_NC
    ),
    compiler_params=pltpu.CompilerParams(
        needs_layout_passes=False, use_tc_tiling_on_sc=False
    ),
    scratch_types=[
        pltpu.VMEM((_NBATCH, _B), jnp.int32),
        pltpu.VMEM((_NBATCH, _B), jnp.int32),
        pltpu.VMEM((_RPT, _H), jnp.float32),
        pltpu.VMEM((_RPT, _H), jnp.float32),
        pltpu.VMEM((_RPT, _W), jnp.float32),
        pltpu.VMEM((_B, _W), jnp.float32),
        pltpu.VMEM((_B, _W), jnp.float32),
        pltpu.VMEM((_B, _W), jnp.float32),
        pltpu.VMEM((_B, _W), jnp.float32),
        pltpu.VMEM((_B, _H), jnp.float32),
        pltpu.VMEM((_B,), jnp.float32),
        pltpu.VMEM((_B, _H), jnp.float32),
        pltpu.VMEM((_B,), jnp.float32),
        pltpu.VMEM((_RPT,), jnp.float32),
        pltpu.VMEM((_RPT,), jnp.float32),
        pltpu.VMEM((_L,), jnp.float32),
        pltpu.VMEM_SHARED((_N1,), jnp.float32),
        pltpu.VMEM_SHARED((_N1, _H), jnp.float32),
    ],
)


def _pre_body(x_ref, w_ref, b_ref, o_ref):
    acc = jnp.dot(x_ref[...], w_ref[...], preferred_element_type=jnp.float32)
    o_ref[...] = jnp.maximum(acc + b_ref[...], jnp.float32(0.0))


_pre = pl.pallas_call(
    _pre_body,
    grid=(10,),
    in_specs=[
        pl.BlockSpec((_N // 10, _D), lambda i: (i, 0)),
        pl.BlockSpec((_D, _H), lambda i: (0, 0)),
        pl.BlockSpec((1, _H), lambda i: (0, 0)),
    ],
    out_specs=pl.BlockSpec((_N // 10, _H), lambda i: (i, 0)),
    out_shape=jax.ShapeDtypeStruct((_N, _H), jnp.float32),
)


def _post_body(n_ref, d_ref, w_ref, b_ref, o_ref):
    nsum = n_ref[0] + n_ref[1]
    dsum = d_ref[0] + d_ref[1]
    h = nsum * (jnp.float32(1.0)
                / jnp.maximum(dsum, jnp.float32(1e-30)))[:, None]
    z = jnp.dot(h, w_ref[...], preferred_element_type=jnp.float32)
    z = z + b_ref[...]
    z = z - jnp.max(z, axis=1, keepdims=True)
    o_ref[...] = z - jnp.log(jnp.sum(jnp.exp(z), axis=1, keepdims=True))


_post = pl.pallas_call(
    _post_body,
    grid=(10,),
    in_specs=[
        pl.BlockSpec((_NC, _N1 // 10, _H), lambda i: (0, i, 0)),
        pl.BlockSpec((_NC, _N1 // 10), lambda i: (0, i)),
        pl.BlockSpec((_H, _C), lambda i: (0, 0)),
        pl.BlockSpec((1, _C), lambda i: (0, 0)),
    ],
    out_specs=pl.BlockSpec((_N1 // 10, _C), lambda i: (i, 0)),
    out_shape=jax.ShapeDtypeStruct((_N1, _C), jnp.float32),
)


def kernel(x, edge_index, W1, b1, beta2, beta3, beta4, W2, b2):
    h0 = _pre(x, W1, b1.reshape(1, _H))
    hp = jnp.concatenate(
        [h0, jnp.zeros((_N1 - _N, _H), jnp.float32)], axis=0
    )
    nparts = jnp.stack([hp, jnp.zeros_like(hp)])
    dparts = jnp.stack(
        [jnp.ones((_N1,), jnp.float32), jnp.zeros((_N1,), jnp.float32)]
    )

    src = edge_index[0].astype(jnp.int32)
    dst = edge_index[1].astype(jnp.int32)
    loop = jnp.arange(_N, dtype=jnp.int32)
    pad = _N + (jnp.arange(_E1 - _E - _N, dtype=jnp.int32) % (_N1 - _N))
    srcs = jnp.concatenate([src, loop, pad]).reshape(_NW, _NBATCH, _B)
    dsts = jnp.concatenate([dst, loop, pad]).reshape(_NW, _NBATCH, _B)

    ones = jnp.ones((_L,), jnp.float32)
    nparts, dparts, _t = _conv(nparts, dparts, srcs, dsts, ones)
    nparts, dparts, _t = _conv(nparts, dparts, srcs, dsts, ones * beta2)
    nparts, dparts, _t = _conv(nparts, dparts, srcs, dsts, ones * beta3)
    nparts, dparts, _t = _conv(nparts, dparts, srcs, dsts, ones * beta4)

    out = _post(nparts, dparts, W2, b2.reshape(1, _C))
    return out[:_N]


# speedup vs baseline: 2.2919x; 1.2449x over previous
import jax
import jax.numpy as jnp
from jax import lax
from jax.experimental import pallas as pl
from jax.experimental.pallas import tpu as pltpu
from jax.experimental.pallas import tpu_sc as plsc

_N = 10000
_E = 320000
_D = 128
_H = 16
_C = 16

_L = 16
_NC = 2
_NT = 16
_NW = _NC * _NT
_RPT = 640
_N1 = _NT * _RPT
_B = 128
_NBATCH = 82
_EPW = _NBATCH * _B
_E1 = _NW * _EPW
_W = 2 * _H


def _rsqrt_newton(v):
    v = jnp.maximum(v, jnp.float32(1e-24))
    i = plsc.bitcast(v, jnp.int32)
    i = jnp.int32(0x5F3759DF) - lax.shift_right_logical(i, 1)
    y = plsc.bitcast(i, jnp.float32)
    for _ in range(3):
        y = y * (jnp.float32(1.5) - jnp.float32(0.5) * v * y * y)
    return y


def _conv_body(np_hbm, dp_hbm, srcs_hbm, dsts_hbm, beta_hbm,
               npo_hbm, dpo_hbm, tbl_hbm, nrm_hbm,
               src_t, dst_t, nbuf, n2buf, cbuf,
               rows_s0, rows_d0, nrm0, rows_s1, rows_d1, nrm1,
               out_rows0, ab0, out_rows1, ab1,
               dbuf, d2buf, beta_v,
               denom_spm, num_spm):
    c = lax.axis_index("c")
    s = lax.axis_index("s")
    wid = c * _NT + s
    base = s * _RPT
    lane = lax.iota(jnp.int32, _L)
    zrow = jnp.zeros((_L,), jnp.float32)

    pltpu.sync_copy(srcs_hbm.at[wid], src_t)
    pltpu.sync_copy(dsts_hbm.at[wid], dst_t)
    pltpu.sync_copy(beta_hbm, beta_v)

    pltpu.sync_copy(np_hbm.at[0, pl.ds(base, _RPT)], nbuf)
    pltpu.sync_copy(np_hbm.at[1, pl.ds(base, _RPT)], n2buf)
    pltpu.sync_copy(dp_hbm.at[0, pl.ds(base, _RPT)], dbuf)
    pltpu.sync_copy(dp_hbm.at[1, pl.ds(base, _RPT)], d2buf)

    def p0(rg, carry):
        off = rg * _L
        dv = dbuf[pl.ds(off, _L)] + d2buf[pl.ds(off, _L)]
        dinv = jnp.float32(1.0) / jnp.maximum(dv, jnp.float32(1e-30))
        acc = zrow
        for u in range(_L):
            r = off + u
            hr = (nbuf[r, :] + n2buf[r, :]) * dinv[u]
            cbuf[r, :] = hr
            acc = jnp.where(lane == u, jnp.sum(hr * hr), acc)
            nbuf[r, :] = zrow
        rinv = _rsqrt_newton(acc)
        for u in range(_L):
            r = off + u
            cbuf[r, :] = cbuf[r, :] * rinv[u]
        d2buf[pl.ds(off, _L)] = acc * rinv
        dbuf[pl.ds(off, _L)] = zrow
        return carry

    lax.fori_loop(0, _RPT // _L, p0, None)
    pltpu.sync_copy(cbuf, tbl_hbm.at[pl.ds(base, _RPT)])
    pltpu.sync_copy(d2buf, nrm_hbm.at[pl.ds(base, _RPT)])
    pltpu.sync_copy(dbuf, denom_spm.at[pl.ds(base, _RPT)])
    pltpu.sync_copy(nbuf, num_spm.at[pl.ds(base, _RPT)])
    plsc.subcore_barrier()

    bufs = ((rows_s0, rows_d0, nrm0, out_rows0, ab0),
            (rows_s1, rows_d1, nrm1, out_rows1, ab1))
    bv = beta_v[...]

    def issue_gathers(i, p, sem):
        rs, rd, nr, _, _ = bufs[p]
        si = src_t.at[i]
        return (pltpu.async_copy(tbl_hbm.at[si], rs, sem),
                pltpu.async_copy(tbl_hbm.at[dst_t.at[i]], rd, sem),
                pltpu.async_copy(nrm_hbm.at[si], nr, sem))

    def process(i, p):
        rs, rd, nr, orows, ab = bufs[p]

        def grp(g, carry):
            acc = zrow
            srows = []
            for u in range(_L):
                e = g * _L + u
                sr = rs[e, :]
                srows.append(sr)
                acc = jnp.where(lane == u, jnp.sum(sr * rd[e, :]), acc)
            off = g * _L
            av = jnp.exp(acc * bv)
            ab[pl.ds(off, _L)] = av
            av2 = av * nr[pl.ds(off, _L)]
            for u in range(_L):
                e = g * _L + u
                orows[e, :] = srows[u] * av2[u]
            return carry

        lax.fori_loop(0, _B // _L, grp, None)

    def issue_scatters(i, p, sem):
        _, _, _, orows, ab = bufs[p]
        di = dst_t.at[i]
        return (pltpu.async_copy(ab, denom_spm.at[di], sem, add=True),
                pltpu.async_copy(orows, num_spm.at[di], sem, add=True))

    def p1_scoped(g0, g1, s0, s1):
        def p1(j, carry):
            b0 = j * 2
            dg0 = issue_gathers(b0, 0, g0)
            dg1 = issue_gathers(b0 + 1, 1, g1)
            for d in dg0:
                d.wait()
            process(b0, 0)
            ds0 = issue_scatters(b0, 0, s0)
            for d in dg1:
                d.wait()
            process(b0 + 1, 1)
            ds1 = issue_scatters(b0 + 1, 1, s1)
            for d in ds0:
                d.wait()
            for d in ds1:
                d.wait()
            return carry

        lax.fori_loop(0, _NBATCH // 2, p1, None)

    pl.run_scoped(p1_scoped,
                  g0=pltpu.SemaphoreType.DMA(()),
                  g1=pltpu.SemaphoreType.DMA(()),
                  s0=pltpu.SemaphoreType.DMA(()),
                  s1=pltpu.SemaphoreType.DMA(()))
    plsc.subcore_barrier()

    pltpu.sync_copy(num_spm.at[pl.ds(base, _RPT)], nbuf)
    pltpu.sync_copy(denom_spm.at[pl.ds(base, _RPT)], dbuf)
    pltpu.sync_copy(nbuf, npo_hbm.at[c, pl.ds(base, _RPT)])
    pltpu.sync_copy(dbuf, dpo_hbm.at[c, pl.ds(base, _RPT)])


_conv = pl.kernel(
    _conv_body,
    out_type=(
        jax.ShapeDtypeStruct((_NC, _N1, _H), jnp.float32),
        jax.ShapeDtypeStruct((_NC, _N1), jnp.float32),
        jax.ShapeDtypeStruct((_N1, _H), jnp.float32),
        jax.ShapeDtypeStruct((_N1,), jnp.float32),
    ),
    mesh=plsc.VectorSubcoreMesh(
        core_axis_name="c", subcore_axis_name="s", num_cores=_NC
    ),
    compiler_params=pltpu.CompilerParams(
        needs_layout_passes=False, use_tc_tiling_on_sc=False
    ),
    scratch_types=[
        pltpu.VMEM((_NBATCH, _B), jnp.int32),
        pltpu.VMEM((_NBATCH, _B), jnp.int32),
        pltpu.VMEM((_RPT, _H), jnp.float32),
        pltpu.VMEM((_RPT, _H), jnp.float32),
        pltpu.VMEM((_RPT, _H), jnp.float32),
        pltpu.VMEM((_B, _H), jnp.float32),
        pltpu.VMEM((_B, _H), jnp.float32),
        pltpu.VMEM((_B,), jnp.float32),
        pltpu.VMEM((_B, _H), jnp.float32),
        pltpu.VMEM((_B, _H), jnp.float32),
        pltpu.VMEM((_B,), jnp.float32),
        pltpu.VMEM((_B, _H), jnp.float32),
        pltpu.VMEM((_B,), jnp.float32),
        pltpu.VMEM((_B, _H), jnp.float32),
        pltpu.VMEM((_B,), jnp.float32),
        pltpu.VMEM((_RPT,), jnp.float32),
        pltpu.VMEM((_RPT,), jnp.float32),
        pltpu.VMEM((_L,), jnp.float32),
        pltpu.VMEM_SHARED((_N1,), jnp.float32),
        pltpu.VMEM_SHARED((_N1, _H), jnp.float32),
    ],
)


def _pre_body(x_ref, w_ref, b_ref, o_ref):
    acc = jnp.dot(x_ref[...], w_ref[...], preferred_element_type=jnp.float32)
    o_ref[...] = jnp.maximum(acc + b_ref[...], jnp.float32(0.0))


_pre = pl.pallas_call(
    _pre_body,
    grid=(10,),
    in_specs=[
        pl.BlockSpec((_N // 10, _D), lambda i: (i, 0)),
        pl.BlockSpec((_D, _H), lambda i: (0, 0)),
        pl.BlockSpec((1, _H), lambda i: (0, 0)),
    ],
    out_specs=pl.BlockSpec((_N // 10, _H), lambda i: (i, 0)),
    out_shape=jax.ShapeDtypeStruct((_N, _H), jnp.float32),
)


def _post_body(n_ref, d_ref, w_ref, b_ref, o_ref):
    nsum = n_ref[0] + n_ref[1]
    dsum = d_ref[0] + d_ref[1]
    h = nsum * (jnp.float32(1.0)
                / jnp.maximum(dsum, jnp.float32(1e-30)))[:, None]
    z = jnp.dot(h, w_ref[...], preferred_element_type=jnp.float32)
    z = z + b_ref[...]
    z = z - jnp.max(z, axis=1, keepdims=True)
    o_ref[...] = z - jnp.log(jnp.sum(jnp.exp(z), axis=1, keepdims=True))


_post = pl.pallas_call(
    _post_body,
    grid=(10,),
    in_specs=[
        pl.BlockSpec((_NC, _N1 // 10, _H), lambda i: (0, i, 0)),
        pl.BlockSpec((_NC, _N1 // 10), lambda i: (0, i)),
        pl.BlockSpec((_H, _C), lambda i: (0, 0)),
        pl.BlockSpec((1, _C), lambda i: (0, 0)),
    ],
    out_specs=pl.BlockSpec((_N1 // 10, _C), lambda i: (i, 0)),
    out_shape=jax.ShapeDtypeStruct((_N1, _C), jnp.float32),
)


def kernel(x, edge_index, W1, b1, beta2, beta3, beta4, W2, b2):
    h0 = _pre(x, W1, b1.reshape(1, _H))
    hp = jnp.concatenate(
        [h0, jnp.zeros((_N1 - _N, _H), jnp.float32)], axis=0
    )
    nparts = jnp.stack([hp, jnp.zeros_like(hp)])
    dparts = jnp.stack(
        [jnp.ones((_N1,), jnp.float32), jnp.zeros((_N1,), jnp.float32)]
    )

    src = edge_index[0].astype(jnp.int32)
    dst = edge_index[1].astype(jnp.int32)
    loop = jnp.arange(_N, dtype=jnp.int32)
    pad = _N + (jnp.arange(_E1 - _E - _N, dtype=jnp.int32) % (_N1 - _N))
    srcs = jnp.concatenate([src, loop, pad]).reshape(_NW, _NBATCH, _B)
    dsts = jnp.concatenate([dst, loop, pad]).reshape(_NW, _NBATCH, _B)

    ones = jnp.ones((_L,), jnp.float32)
    nparts, dparts, _t, _n = _conv(nparts, dparts, srcs, dsts, ones)
    nparts, dparts, _t, _n = _conv(nparts, dparts, srcs, dsts, ones * beta2)
    nparts, dparts, _t, _n = _conv(nparts, dparts, srcs, dsts, ones * beta3)
    nparts, dparts, _t, _n = _conv(nparts, dparts, srcs, dsts, ones * beta4)

    out = _post(nparts, dparts, W2, b2.reshape(1, _C))
    return out[:_N]


# speedup vs baseline: 2.3985x; 1.0465x over previous
import jax
import jax.numpy as jnp
from jax import lax
from jax.experimental import pallas as pl
from jax.experimental.pallas import tpu as pltpu
from jax.experimental.pallas import tpu_sc as plsc

_N = 10000
_E = 320000
_D = 128
_H = 16
_C = 16

_L = 16
_NC = 2
_NT = 16
_NW = _NC * _NT
_RPT = 640
_N1 = _NT * _RPT
_B = 128
_NBATCH = 82
_EPW = _NBATCH * _B
_E1 = _NW * _EPW
_W = 2 * _H


def _rsqrt_newton(v):
    v = jnp.maximum(v, jnp.float32(1e-24))
    i = plsc.bitcast(v, jnp.int32)
    i = jnp.int32(0x5F3759DF) - lax.shift_right_logical(i, 1)
    y = plsc.bitcast(i, jnp.float32)
    for _ in range(3):
        y = y * (jnp.float32(1.5) - jnp.float32(0.5) * v * y * y)
    return y


def _conv_body(np_hbm, dp_hbm, srcs_hbm, dsts_hbm, beta_hbm,
               npo_hbm, dpo_hbm, tbl_hbm, nrm_hbm,
               src_t, dst_t, nbuf, n2buf, cbuf,
               rows_s0, rows_d0, nrm0, rows_s1, rows_d1, nrm1,
               out_rows0, ab0, out_rows1, ab1,
               dbuf, d2buf, beta_v,
               denom_spm, num_spm):
    c = lax.axis_index("c")
    s = lax.axis_index("s")
    wid = c * _NT + s
    base = s * _RPT
    lane = lax.iota(jnp.int32, _L)
    zrow = jnp.zeros((_L,), jnp.float32)

    pltpu.sync_copy(srcs_hbm.at[wid], src_t)
    pltpu.sync_copy(dsts_hbm.at[wid], dst_t)
    pltpu.sync_copy(beta_hbm, beta_v)

    pltpu.sync_copy(np_hbm.at[0, pl.ds(base, _RPT)], nbuf)
    pltpu.sync_copy(np_hbm.at[1, pl.ds(base, _RPT)], n2buf)
    pltpu.sync_copy(dp_hbm.at[0, pl.ds(base, _RPT)], dbuf)
    pltpu.sync_copy(dp_hbm.at[1, pl.ds(base, _RPT)], d2buf)

    def p0(rg, carry):
        off = rg * _L
        dv = dbuf[pl.ds(off, _L)] + d2buf[pl.ds(off, _L)]
        dinv = jnp.float32(1.0) / jnp.maximum(dv, jnp.float32(1e-30))
        acc = zrow
        for u in range(_L):
            r = off + u
            hr = (nbuf[r, :] + n2buf[r, :]) * dinv[u]
            cbuf[r, :] = hr
            acc = jnp.where(lane == u, jnp.sum(hr * hr), acc)
            nbuf[r, :] = zrow
        rinv = _rsqrt_newton(acc)
        for u in range(_L):
            r = off + u
            cbuf[r, :] = cbuf[r, :] * rinv[u]
        d2buf[pl.ds(off, _L)] = acc * rinv
        dbuf[pl.ds(off, _L)] = zrow
        return carry

    lax.fori_loop(0, _RPT // _L, p0, None)
    pltpu.sync_copy(cbuf, tbl_hbm.at[pl.ds(base, _RPT)])
    pltpu.sync_copy(d2buf, nrm_hbm.at[pl.ds(base, _RPT)])
    pltpu.sync_copy(dbuf, denom_spm.at[pl.ds(base, _RPT)])
    pltpu.sync_copy(nbuf, num_spm.at[pl.ds(base, _RPT)])
    plsc.subcore_barrier()

    bufs = ((rows_s0, rows_d0, nrm0, out_rows0, ab0),
            (rows_s1, rows_d1, nrm1, out_rows1, ab1))
    bv = beta_v[...]

    def issue_gathers(i, p, sem):
        rs, rd, nr, _, _ = bufs[p]
        si = src_t.at[i]
        return (pltpu.async_copy(tbl_hbm.at[si], rs, sem),
                pltpu.async_copy(tbl_hbm.at[dst_t.at[i]], rd, sem),
                pltpu.async_copy(nrm_hbm.at[si], nr, sem))

    def process(i, p):
        rs, rd, nr, orows, ab = bufs[p]

        @plsc.parallel_loop(0, _B // _L, unroll=2)
        def grp(g):
            acc = zrow
            srows = []
            for u in range(_L):
                e = g * _L + u
                sr = rs[e, :]
                srows.append(sr)
                acc = jnp.where(lane == u, jnp.sum(sr * rd[e, :]), acc)
            off = g * _L
            av = jnp.exp(acc * bv)
            ab[pl.ds(off, _L)] = av
            av2 = av * nr[pl.ds(off, _L)]
            for u in range(_L):
                e = g * _L + u
                orows[e, :] = srows[u] * av2[u]

    def issue_scatters(i, p, sem):
        _, _, _, orows, ab = bufs[p]
        di = dst_t.at[i]
        return (pltpu.async_copy(ab, denom_spm.at[di], sem, add=True),
                pltpu.async_copy(orows, num_spm.at[di], sem, add=True))

    def p1_scoped(g0, g1, s0, s1):
        def p1(j, carry):
            b0 = j * 2
            dg0 = issue_gathers(b0, 0, g0)
            dg1 = issue_gathers(b0 + 1, 1, g1)
            for d in dg0:
                d.wait()
            process(b0, 0)
            ds0 = issue_scatters(b0, 0, s0)
            for d in dg1:
                d.wait()
            process(b0 + 1, 1)
            ds1 = issue_scatters(b0 + 1, 1, s1)
            for d in ds0:
                d.wait()
            for d in ds1:
                d.wait()
            return carry

        lax.fori_loop(0, _NBATCH // 2, p1, None)

    pl.run_scoped(p1_scoped,
                  g0=pltpu.SemaphoreType.DMA(()),
                  g1=pltpu.SemaphoreType.DMA(()),
                  s0=pltpu.SemaphoreType.DMA(()),
                  s1=pltpu.SemaphoreType.DMA(()))
    plsc.subcore_barrier()

    pltpu.sync_copy(num_spm.at[pl.ds(base, _RPT)], nbuf)
    pltpu.sync_copy(denom_spm.at[pl.ds(base, _RPT)], dbuf)
    pltpu.sync_copy(nbuf, npo_hbm.at[c, pl.ds(base, _RPT)])
    pltpu.sync_copy(dbuf, dpo_hbm.at[c, pl.ds(base, _RPT)])


_conv = pl.kernel(
    _conv_body,
    out_type=(
        jax.ShapeDtypeStruct((_NC, _N1, _H), jnp.float32),
        jax.ShapeDtypeStruct((_NC, _N1), jnp.float32),
        jax.ShapeDtypeStruct((_N1, _H), jnp.float32),
        jax.ShapeDtypeStruct((_N1,), jnp.float32),
    ),
    mesh=plsc.VectorSubcoreMesh(
        core_axis_name="c", subcore_axis_name="s", num_cores=_NC
    ),
    compiler_params=pltpu.CompilerParams(
        needs_layout_passes=False, use_tc_tiling_on_sc=False
    ),
    scratch_types=[
        pltpu.VMEM((_NBATCH, _B), jnp.int32),
        pltpu.VMEM((_NBATCH, _B), jnp.int32),
        pltpu.VMEM((_RPT, _H), jnp.float32),
        pltpu.VMEM((_RPT, _H), jnp.float32),
        pltpu.VMEM((_RPT, _H), jnp.float32),
        pltpu.VMEM((_B, _H), jnp.float32),
        pltpu.VMEM((_B, _H), jnp.float32),
        pltpu.VMEM((_B,), jnp.float32),
        pltpu.VMEM((_B, _H), jnp.float32),
        pltpu.VMEM((_B, _H), jnp.float32),
        pltpu.VMEM((_B,), jnp.float32),
        pltpu.VMEM((_B, _H), jnp.float32),
        pltpu.VMEM((_B,), jnp.float32),
        pltpu.VMEM((_B, _H), jnp.float32),
        pltpu.VMEM((_B,), jnp.float32),
        pltpu.VMEM((_RPT,), jnp.float32),
        pltpu.VMEM((_RPT,), jnp.float32),
        pltpu.VMEM((_L,), jnp.float32),
        pltpu.VMEM_SHARED((_N1,), jnp.float32),
        pltpu.VMEM_SHARED((_N1, _H), jnp.float32),
    ],
)


def _pre_body(x_ref, w_ref, b_ref, o_ref):
    acc = jnp.dot(x_ref[...], w_ref[...], preferred_element_type=jnp.float32)
    o_ref[...] = jnp.maximum(acc + b_ref[...], jnp.float32(0.0))


_pre = pl.pallas_call(
    _pre_body,
    grid=(10,),
    in_specs=[
        pl.BlockSpec((_N // 10, _D), lambda i: (i, 0)),
        pl.BlockSpec((_D, _H), lambda i: (0, 0)),
        pl.BlockSpec((1, _H), lambda i: (0, 0)),
    ],
    out_specs=pl.BlockSpec((_N // 10, _H), lambda i: (i, 0)),
    out_shape=jax.ShapeDtypeStruct((_N, _H), jnp.float32),
)


def _post_body(n_ref, d_ref, w_ref, b_ref, o_ref):
    nsum = n_ref[0] + n_ref[1]
    dsum = d_ref[0] + d_ref[1]
    h = nsum * (jnp.float32(1.0)
                / jnp.maximum(dsum, jnp.float32(1e-30)))[:, None]
    z = jnp.dot(h, w_ref[...], preferred_element_type=jnp.float32)
    z = z + b_ref[...]
    z = z - jnp.max(z, axis=1, keepdims=True)
    o_ref[...] = z - jnp.log(jnp.sum(jnp.exp(z), axis=1, keepdims=True))


_post = pl.pallas_call(
    _post_body,
    grid=(10,),
    in_specs=[
        pl.BlockSpec((_NC, _N1 // 10, _H), lambda i: (0, i, 0)),
        pl.BlockSpec((_NC, _N1 // 10), lambda i: (0, i)),
        pl.BlockSpec((_H, _C), lambda i: (0, 0)),
        pl.BlockSpec((1, _C), lambda i: (0, 0)),
    ],
    out_specs=pl.BlockSpec((_N1 // 10, _C), lambda i: (i, 0)),
    out_shape=jax.ShapeDtypeStruct((_N1, _C), jnp.float32),
)


def kernel(x, edge_index, W1, b1, beta2, beta3, beta4, W2, b2):
    h0 = _pre(x, W1, b1.reshape(1, _H))
    hp = jnp.concatenate(
        [h0, jnp.zeros((_N1 - _N, _H), jnp.float32)], axis=0
    )
    nparts = jnp.stack([hp, jnp.zeros_like(hp)])
    dparts = jnp.stack(
        [jnp.ones((_N1,), jnp.float32), jnp.zeros((_N1,), jnp.float32)]
    )

    src = edge_index[0].astype(jnp.int32)
    dst = edge_index[1].astype(jnp.int32)
    loop = jnp.arange(_N, dtype=jnp.int32)
    pad = _N + (jnp.arange(_E1 - _E - _N, dtype=jnp.int32) % (_N1 - _N))
    srcs = jnp.concatenate([src, loop, pad]).reshape(_NW, _NBATCH, _B)
    dsts = jnp.concatenate([dst, loop, pad]).reshape(_NW, _NBATCH, _B)

    ones = jnp.ones((_L,), jnp.float32)
    nparts, dparts, _t, _n = _conv(nparts, dparts, srcs, dsts, ones)
    nparts, dparts, _t, _n = _conv(nparts, dparts, srcs, dsts, ones * beta2)
    nparts, dparts, _t, _n = _conv(nparts, dparts, srcs, dsts, ones * beta3)
    nparts, dparts, _t, _n = _conv(nparts, dparts, srcs, dsts, ones * beta4)

    out = _post(nparts, dparts, W2, b2.reshape(1, _C))
    return out[:_N]
